# baseline (speedup 1.0000x reference)
import functools

import jax
import jax.numpy as jnp
from jax import lax
from jax.experimental import pallas as pl
from jax.experimental.pallas import tpu as pltpu
from jax.experimental.pallas import tpu_sc as plsc

BATCH = 1024
MPD = 50
EMB = 128
LANES = 16
NC, NS = 2, 16
NW = NC * NS
DOCS_PER_W = BATCH // NW
PAIR = 2
PAIRS_PER_W = DOCS_PER_W // PAIR
NBUF = 8


def _sc_embedding_bag(idx, table, bias):
    mesh = plsc.VectorSubcoreMesh(core_axis_name="c", subcore_axis_name="s")

    @functools.partial(
        pl.kernel,
        out_type=jax.ShapeDtypeStruct((BATCH, EMB), jnp.float32),
        mesh=mesh,
        scratch_types=[
            pltpu.VMEM((PAIRS_PER_W, PAIR * MPD), jnp.int32),
        ]
        + [pltpu.VMEM((PAIR * MPD, EMB), jnp.float32)] * NBUF
        + [
            pltpu.VMEM((DOCS_PER_W, EMB), jnp.float32),
            pltpu.VMEM((EMB,), jnp.float32),
        ]
        + [pltpu.SemaphoreType.DMA] * NBUF,
    )
    def kern(idx_hbm, tab_hbm, b_hbm, out_hbm, idx_v, *rest):
        rows_bufs = rest[:NBUF]
        out_v, bias_v = rest[NBUF], rest[NBUF + 1]
        sems = rest[NBUF + 2:]
        wid = lax.axis_index("s") * NC + lax.axis_index("c")
        base = wid * DOCS_PER_W
        pltpu.sync_copy(b_hbm, bias_v)
        pltpu.sync_copy(idx_hbm.at[pl.ds(wid * PAIRS_PER_W, PAIRS_PER_W)], idx_v)

        for j in range(NBUF):
            pltpu.async_copy(tab_hbm.at[idx_v.at[j]], rows_bufs[j], sems[j])

        @pl.loop(0, PAIRS_PER_W, step=NBUF)
        def _pair(p0):
            for j in range(NBUF):
                p = p0 + j
                rows = rows_bufs[j]
                pltpu.make_async_copy(
                    tab_hbm.at[idx_v.at[p]], rows, sems[j]).wait()
                for sub in range(PAIR):
                    accs0 = tuple(bias_v[pl.ds(c * LANES, LANES)]
                                  for c in range(EMB // LANES))

                    def body(r, accs):
                        return tuple(
                            accs[c] + rows[r, pl.ds(c * LANES, LANES)]
                            for c in range(EMB // LANES))

                    accs = plsc.parallel_loop(
                        sub * MPD, (sub + 1) * MPD, 1, unroll=5,
                        carry=accs0)(body)
                    d = p * PAIR + sub
                    for c in range(EMB // LANES):
                        out_v[d, pl.ds(c * LANES, LANES)] = jnp.maximum(
                            accs[c], 0.0)

                @pl.when(p + NBUF < PAIRS_PER_W)
                def _():
                    pltpu.async_copy(
                        tab_hbm.at[idx_v.at[p + NBUF]], rows, sems[j])

        pltpu.sync_copy(out_v, out_hbm.at[pl.ds(base, DOCS_PER_W)])

    return kern(idx, table, bias)


def kernel(document_mention_indices, W, b):
    idx = document_mention_indices.astype(jnp.int32).reshape(
        BATCH // PAIR, PAIR * MPD)
    table = W.T
    return _sc_embedding_bag(idx, table, b)


# speedup vs baseline: 1.0170x; 1.0170x over previous
import functools

import jax
import jax.numpy as jnp
from jax import lax
from jax.experimental import pallas as pl
from jax.experimental.pallas import tpu as pltpu
from jax.experimental.pallas import tpu_sc as plsc

BATCH = 1024
MPD = 50
EMB = 128
LANES = 16
NC, NS = 2, 16
NW = NC * NS
DOCS_PER_W = BATCH // NW
PAIR = 2
PAIRS_PER_W = DOCS_PER_W // PAIR
NBUF = 4


def _sc_embedding_bag(idx, table, bias):
    mesh = plsc.VectorSubcoreMesh(core_axis_name="c", subcore_axis_name="s")

    @functools.partial(
        pl.kernel,
        out_type=jax.ShapeDtypeStruct((BATCH, EMB), jnp.float32),
        mesh=mesh,
        scratch_types=[
            pltpu.VMEM((PAIRS_PER_W, PAIR * MPD), jnp.int32),
        ]
        + [pltpu.VMEM((PAIR * MPD, EMB), jnp.float32)] * NBUF
        + [
            pltpu.VMEM((DOCS_PER_W, EMB), jnp.float32),
            pltpu.VMEM((EMB,), jnp.float32),
        ]
        + [pltpu.SemaphoreType.DMA] * NBUF,
    )
    def kern(idx_hbm, tab_hbm, b_hbm, out_hbm, idx_v, *rest):
        rows_bufs = rest[:NBUF]
        out_v, bias_v = rest[NBUF], rest[NBUF + 1]
        sems = rest[NBUF + 2:]
        wid = lax.axis_index("s") * NC + lax.axis_index("c")
        base = wid * DOCS_PER_W
        pltpu.sync_copy(b_hbm, bias_v)
        pltpu.sync_copy(idx_hbm.at[pl.ds(wid * PAIRS_PER_W, PAIRS_PER_W)], idx_v)

        for j in range(NBUF):
            pltpu.async_copy(tab_hbm.at[idx_v.at[j]], rows_bufs[j], sems[j])

        @pl.loop(0, PAIRS_PER_W, step=NBUF)
        def _pair(p0):
            for j in range(NBUF):
                p = p0 + j
                rows = rows_bufs[j]
                pltpu.make_async_copy(
                    tab_hbm.at[idx_v.at[p]], rows, sems[j]).wait()
                for sub in range(PAIR):
                    accs0 = tuple(bias_v[pl.ds(c * LANES, LANES)]
                                  for c in range(EMB // LANES))

                    def body(r, accs):
                        return tuple(
                            accs[c] + rows[r, pl.ds(c * LANES, LANES)]
                            for c in range(EMB // LANES))

                    accs = plsc.parallel_loop(
                        sub * MPD, (sub + 1) * MPD, 1, unroll=5,
                        carry=accs0)(body)
                    d = p * PAIR + sub
                    for c in range(EMB // LANES):
                        out_v[d, pl.ds(c * LANES, LANES)] = jnp.maximum(
                            accs[c], 0.0)

                @pl.when(p + NBUF < PAIRS_PER_W)
                def _():
                    pltpu.async_copy(
                        tab_hbm.at[idx_v.at[p + NBUF]], rows, sems[j])

        pltpu.sync_copy(out_v, out_hbm.at[pl.ds(base, DOCS_PER_W)])

    return kern(idx, table, bias)


def kernel(document_mention_indices, W, b):
    idx = document_mention_indices.astype(jnp.int32).reshape(
        BATCH // PAIR, PAIR * MPD)
    table = W.T
    return _sc_embedding_bag(idx, table, b)
